# Initial kernel scaffold; baseline (speedup 1.0000x reference)
import jax
import jax.numpy as jnp
from jax.experimental import pallas as pl


def kernel(x, w2v_weight):
    raise NotImplementedError("write your pallas kernel here")


# speedup vs baseline: 1.0073x; 1.0073x over previous
import functools

import jax
import jax.numpy as jnp
from jax import lax
from jax.experimental import pallas as pl
from jax.experimental.pallas import tpu as pltpu
from jax.experimental.pallas import tpu_sc as plsc

NUM_CLASSES = 1000
EMBED_DIM = 999
BATCH = 4096
HIST = 20

NUM_ROWS = BATCH * HIST
PAD_DIM = 1000
NC, NS = 2, 16
NW = NC * NS
ROWS_PER_W = NUM_ROWS // NW
CHUNK = 40
NCHUNK = ROWS_PER_W // CHUNK

_MESH = plsc.VectorSubcoreMesh(
    core_axis_name="c", subcore_axis_name="s", num_cores=NC, num_subcores=NS
)


def _embed_sc_body(idx_hbm, table_hbm, out_hbm, idx_v, rows0, rows1, sem0, sem1):
    wid = lax.axis_index("s") * NC + lax.axis_index("c")
    base = wid * ROWS_PER_W
    pltpu.sync_copy(idx_hbm.at[wid], idx_v)

    def gather(j, buf, sem):
        pltpu.async_copy(table_hbm.at[idx_v.at[j]], buf, sem)

    def drain_and_write(j, buf, sem):
        pltpu.make_async_copy(table_hbm.at[idx_v.at[j]], buf, sem).wait()
        pltpu.sync_copy(buf, out_hbm.at[pl.ds(base + j * CHUNK, CHUNK)])

    gather(0, rows0, sem0)

    def body(it, carry):
        j0 = 2 * it
        gather(j0 + 1, rows1, sem1)
        drain_and_write(j0, rows0, sem0)

        @pl.when(it + 1 < NCHUNK // 2)
        def _prefetch():
            gather(j0 + 2, rows0, sem0)

        drain_and_write(j0 + 1, rows1, sem1)
        return carry

    lax.fori_loop(0, NCHUNK // 2, body, 0)


_embed_sc = pl.kernel(
    _embed_sc_body,
    out_type=jax.ShapeDtypeStruct((NUM_ROWS, PAD_DIM), jnp.float32),
    mesh=_MESH,
    scratch_types=[
        pltpu.VMEM((NCHUNK, CHUNK), jnp.int32),
        pltpu.VMEM((CHUNK, PAD_DIM), jnp.float32),
        pltpu.VMEM((CHUNK, PAD_DIM), jnp.float32),
        pltpu.SemaphoreType.DMA,
        pltpu.SemaphoreType.DMA,
    ],
    compiler_params=pltpu.CompilerParams(use_tc_tiling_on_sc=False),
)


def kernel(x, w2v_weight):
    idx = x.astype(jnp.int32).reshape(NW, NCHUNK, CHUNK)
    table = jnp.pad(w2v_weight, ((0, 0), (0, PAD_DIM - EMBED_DIM)))
    out = _embed_sc(idx, table)
    return out[:, :EMBED_DIM].reshape(BATCH, HIST, EMBED_DIM)
